# Initial kernel scaffold; baseline (speedup 1.0000x reference)
import jax
import jax.numpy as jnp
from jax.experimental import pallas as pl


def kernel(ev_features, cs_features, tr_features, env_features, edge_index, ev_indexes, cs_indexes, tr_indexes, env_indexes, W_ev, b_ev, W_cs, b_cs, W_tr, b_tr, W_env, b_env, W_g1, b_g1, W_g2, b_g2, W_gl, b_gl):
    raise NotImplementedError("write your pallas kernel here")


# speedup vs baseline: 13.8042x; 13.8042x over previous
import functools

import jax
import jax.numpy as jnp
from jax import lax
from jax.experimental import pallas as pl
from jax.experimental.pallas import tpu as pltpu
from jax.experimental.pallas import tpu_sc as plsc

N_EV, N_CS, N_TR, N_ENV = 60000, 20000, 15000, 5000
NN = N_EV + N_CS + N_TR + N_ENV
NE = 1600000
FD, HD = 8, 32
MAX_ACTION = 1.0

NC, NS = 2, 16
NWORK = NC * NS
EPW = NE // NWORK
CHUNK = 80
NCHUNK = EPW // CHUNK
NP = 100096
STRIPE = NP // NS

BN = 5000
GRID = NN // BN

_MESH = plsc.VectorSubcoreMesh(
    core_axis_name="c", subcore_axis_name="s", num_cores=NC, num_subcores=NS)


def _agg_body(src_hbm, dst_hbm, u_hbm, zeros_hbm, out_hbm,
              sidx, didx, rows, acc, sem):
  c = lax.axis_index("c")
  s = lax.axis_index("s")
  wid = s * NC + c
  base = wid * EPW
  row0 = s * STRIPE
  pltpu.sync_copy(zeros_hbm.at[pl.ds(row0, STRIPE)],
                  acc.at[pl.ds(row0, STRIPE)])
  plsc.subcore_barrier()

  def step(j, carry):
    off = pl.multiple_of(base + j * CHUNK, 8)
    pltpu.sync_copy(src_hbm.at[pl.ds(off, CHUNK)], sidx)
    pltpu.sync_copy(dst_hbm.at[pl.ds(off, CHUNK)], didx)
    pltpu.async_copy(u_hbm.at[sidx], rows, sem).wait()
    pltpu.sync_copy(rows, acc.at[didx], add=True)
    return carry

  lax.fori_loop(0, NCHUNK, step, 0)
  plsc.subcore_barrier()
  pltpu.sync_copy(acc.at[pl.ds(row0, STRIPE)],
                  out_hbm.at[c, pl.ds(row0, STRIPE)])


_agg_call = pl.kernel(
    _agg_body,
    out_type=jax.ShapeDtypeStruct((NC, NP, FD), jnp.float32),
    mesh=_MESH,
    scratch_types=[
        pltpu.VMEM((CHUNK,), jnp.int32),
        pltpu.VMEM((CHUNK,), jnp.int32),
        pltpu.VMEM((CHUNK, FD), jnp.float32),
        pltpu.VMEM_SHARED((NP, FD), jnp.float32),
        pltpu.SemaphoreType.DMA,
    ],
    compiler_params=pltpu.CompilerParams(use_tc_tiling_on_sc=False),
    name="sc_edge_agg",
)


def _deg_body(dst_hbm, ones_hbm, zeros_hbm, out_hbm, didx, ones_v, acc):
  c = lax.axis_index("c")
  s = lax.axis_index("s")
  wid = s * NC + c
  base = wid * EPW
  row0 = s * STRIPE
  pltpu.sync_copy(ones_hbm, ones_v)
  pltpu.sync_copy(zeros_hbm.at[pl.ds(row0, STRIPE)],
                  acc.at[pl.ds(row0, STRIPE)])
  plsc.subcore_barrier()

  def step(j, carry):
    off = pl.multiple_of(base + j * CHUNK, 8)
    pltpu.sync_copy(dst_hbm.at[pl.ds(off, CHUNK)], didx)
    pltpu.sync_copy(ones_v, acc.at[didx], add=True)
    return carry

  lax.fori_loop(0, NCHUNK, step, 0)
  plsc.subcore_barrier()
  pltpu.sync_copy(acc.at[pl.ds(row0, STRIPE)],
                  out_hbm.at[c, pl.ds(row0, STRIPE)])


_deg_call = pl.kernel(
    _deg_body,
    out_type=jax.ShapeDtypeStruct((NC, NP, FD), jnp.float32),
    mesh=_MESH,
    scratch_types=[
        pltpu.VMEM((CHUNK,), jnp.int32),
        pltpu.VMEM((CHUNK, FD), jnp.float32),
        pltpu.VMEM_SHARED((NP, FD), jnp.float32),
    ],
    compiler_params=pltpu.CompilerParams(use_tc_tiling_on_sc=False),
    name="sc_degree",
)


def _blk_type(i):
  i = jnp.asarray(i)
  return ((i >= N_EV // BN).astype(jnp.int32)
          + (i >= (N_EV + N_CS) // BN).astype(jnp.int32)
          + (i >= (N_EV + N_CS + N_TR) // BN).astype(jnp.int32))


def _tc_embed_body(feat_ref, w_ref, b_ref, d_ref, dinv_ref, u1_ref):
  x = feat_ref[...]
  w = w_ref[0]
  b = b_ref[0]
  emb = jnp.dot(x, w, preferred_element_type=jnp.float32) + b
  x0 = jnp.maximum(emb, 0.0)
  d = d_ref[...]
  deg = d[0, :, 0:1] + d[1, :, 0:1] + 1.0
  dinv = lax.rsqrt(jnp.maximum(deg, 1.0))
  dinv_ref[...] = dinv
  u1_ref[...] = x0 * dinv


def _tc_dense1_body(p_ref, u1_ref, dinv_ref, w1_ref, b1_ref, w2_ref, u2_ref):
  p = p_ref[...]
  dinv = dinv_ref[...]
  agg = (p[0] + p[1] + u1_ref[...]) * dinv
  x1 = jnp.maximum(
      jnp.dot(agg, w1_ref[...], preferred_element_type=jnp.float32)
      + b1_ref[...], 0.0)
  u2_ref[...] = jnp.dot(
      x1, w2_ref[...], preferred_element_type=jnp.float32) * dinv


def _tc_dense2_body(q_ref, u2_ref, dinv_ref, b2_ref, wl_ref, u3_ref):
  q = q_ref[...]
  dinv = dinv_ref[...]
  x2 = jnp.maximum((q[0] + q[1] + u2_ref[...]) * dinv + b2_ref[...], 0.0)
  h3 = jnp.sum(x2 * wl_ref[...], axis=1, keepdims=True)
  col0 = (lax.broadcasted_iota(jnp.int32, (1, FD), 1) == 0)
  u3_ref[...] = (h3 * dinv) * col0.astype(jnp.float32)


def _tc_final_body(r_ref, u3_ref, dinv_ref, bl_ref, out_ref):
  r = r_ref[...]
  v = (r[0, :, 0:1] + r[1, :, 0:1] + u3_ref[:, 0:1]) * dinv_ref[...]
  out_ref[...] = MAX_ACTION * jnp.tanh(v + bl_ref[...])


def _full(shape):
  return pl.BlockSpec(shape, lambda i: (0,) * len(shape))


_row8 = pl.BlockSpec((BN, FD), lambda i: (i, 0))
_row1 = pl.BlockSpec((BN, 1), lambda i: (i, 0))
_par8 = pl.BlockSpec((NC, BN, FD), lambda i: (0, i, 0))

_tc_embed = pl.pallas_call(
    _tc_embed_body,
    grid=(GRID,),
    in_specs=[
        pl.BlockSpec((BN, 16), lambda i: (i, 0)),
        pl.BlockSpec((1, 16, FD), lambda i: (_blk_type(i), 0, 0)),
        pl.BlockSpec((1, 1, FD), lambda i: (_blk_type(i), 0, 0)),
        _par8,
    ],
    out_specs=[_row1, _row8],
    out_shape=[
        jax.ShapeDtypeStruct((NN, 1), jnp.float32),
        jax.ShapeDtypeStruct((NN, FD), jnp.float32),
    ],
)

_tc_dense1 = pl.pallas_call(
    _tc_dense1_body,
    grid=(GRID,),
    in_specs=[_par8, _row8, _row1, _full((FD, HD)), _full((1, HD)),
              _full((HD, FD))],
    out_specs=_row8,
    out_shape=jax.ShapeDtypeStruct((NN, FD), jnp.float32),
)

_tc_dense2 = pl.pallas_call(
    _tc_dense2_body,
    grid=(GRID,),
    in_specs=[_par8, _row8, _row1, _full((1, FD)), _full((1, FD))],
    out_specs=_row8,
    out_shape=jax.ShapeDtypeStruct((NN, FD), jnp.float32),
)

_tc_final = pl.pallas_call(
    _tc_final_body,
    grid=(GRID,),
    in_specs=[_par8, _row8, _row1, _full((1, 1))],
    out_specs=_row1,
    out_shape=jax.ShapeDtypeStruct((NN, 1), jnp.float32),
)


def kernel(ev_features, cs_features, tr_features, env_features, edge_index,
           ev_indexes, cs_indexes, tr_indexes, env_indexes,
           W_ev, b_ev, W_cs, b_cs, W_tr, b_tr, W_env, b_env,
           W_g1, b_g1, W_g2, b_g2, W_gl, b_gl):
  src = edge_index[0]
  dst = edge_index[1]

  feat = jnp.concatenate([
      ev_features,
      jnp.pad(cs_features, ((0, 0), (0, 4))),
      jnp.pad(tr_features, ((0, 0), (0, 6))),
      jnp.pad(env_features, ((0, 0), (0, 8))),
  ], axis=0)
  wstack = jnp.stack([
      W_ev,
      jnp.pad(W_cs, ((0, 4), (0, 0))),
      jnp.pad(W_tr, ((0, 6), (0, 0))),
      jnp.pad(W_env, ((0, 8), (0, 0))),
  ])
  bstack = jnp.stack([b_ev, b_cs, b_tr, b_env])[:, None, :]

  zeros = jnp.zeros((NP, FD), jnp.float32)
  ones_rows = jnp.zeros((CHUNK, FD), jnp.float32).at[:, 0].set(1.0)

  degp = _deg_call(dst, ones_rows, zeros)
  dinv, u1 = _tc_embed(feat, wstack, bstack, degp)
  p = _agg_call(src, dst, u1, zeros)
  u2 = _tc_dense1(p, u1, dinv, W_g1, b_g1[None, :], W_g2)
  q = _agg_call(src, dst, u2, zeros)
  u3 = _tc_dense2(q, u2, dinv, b_g2[None, :], W_gl.reshape(1, FD))
  r = _agg_call(src, dst, u3, zeros)
  out = _tc_final(r, u3, dinv, b_gl.reshape(1, 1))
  return out.reshape(-1)


# speedup vs baseline: 47.8483x; 3.4662x over previous
import functools

import jax
import jax.numpy as jnp
from jax import lax
from jax.experimental import pallas as pl
from jax.experimental.pallas import tpu as pltpu
from jax.experimental.pallas import tpu_sc as plsc

N_EV, N_CS, N_TR, N_ENV = 60000, 20000, 15000, 5000
NN = N_EV + N_CS + N_TR + N_ENV
NE = 1600000
FD, HD = 8, 32
MAX_ACTION = 1.0

NC, NS = 2, 16
NWORK = NC * NS
EPW = NE // NWORK
CHUNK = 80
NCHUNK = EPW // CHUNK
G = 25
NP = 100096
STRIPE = NP // NS

BN = 5000
GRID = NN // BN

_MESH = plsc.VectorSubcoreMesh(
    core_axis_name="c", subcore_axis_name="s", num_cores=NC, num_subcores=NS)


def _agg_body(src2_hbm, dst2_hbm, u_hbm, zeros_hbm, out_hbm,
              sidx, didx, rows, acc, isem, gsem, ssem):
  c = lax.axis_index("c")
  s = lax.axis_index("s")
  wid = s * NC + c
  rowbase0 = wid * NCHUNK
  row0 = s * STRIPE
  pltpu.sync_copy(zeros_hbm.at[pl.ds(row0, STRIPE)],
                  acc.at[pl.ds(row0, STRIPE)])
  plsc.subcore_barrier()

  def step(k, carry):
    rb = rowbase0 + k * G
    ds_ = pltpu.async_copy(src2_hbm.at[pl.ds(rb, G)], sidx, isem)
    dd_ = pltpu.async_copy(dst2_hbm.at[pl.ds(rb, G)], didx, isem)
    ds_.wait()
    dd_.wait()
    gd = [pltpu.async_copy(u_hbm.at[sidx.at[g]], rows.at[g], gsem)
          for g in range(G)]
    for d in gd:
      d.wait()
    sd = [pltpu.async_copy(rows.at[g], acc.at[didx.at[g]], ssem, add=True)
          for g in range(G)]
    for d in sd:
      d.wait()
    return carry

  lax.fori_loop(0, NCHUNK // G, step, 0)
  plsc.subcore_barrier()
  pltpu.sync_copy(acc.at[pl.ds(row0, STRIPE)],
                  out_hbm.at[c, pl.ds(row0, STRIPE)])


_agg_call = pl.kernel(
    _agg_body,
    out_type=jax.ShapeDtypeStruct((NC, NP, FD), jnp.float32),
    mesh=_MESH,
    scratch_types=[
        pltpu.VMEM((G, CHUNK), jnp.int32),
        pltpu.VMEM((G, CHUNK), jnp.int32),
        pltpu.VMEM((G, CHUNK, FD), jnp.float32),
        pltpu.VMEM_SHARED((NP, FD), jnp.float32),
        pltpu.SemaphoreType.DMA,
        pltpu.SemaphoreType.DMA,
        pltpu.SemaphoreType.DMA,
    ],
    compiler_params=pltpu.CompilerParams(use_tc_tiling_on_sc=False),
    name="sc_edge_agg",
)


def _deg_body(dst2_hbm, ones_hbm, zeros_hbm, out_hbm, didx, ones_v, acc,
              isem, ssem):
  c = lax.axis_index("c")
  s = lax.axis_index("s")
  wid = s * NC + c
  rowbase0 = wid * NCHUNK
  row0 = s * STRIPE
  pltpu.sync_copy(ones_hbm, ones_v)
  pltpu.sync_copy(zeros_hbm.at[pl.ds(row0, STRIPE)],
                  acc.at[pl.ds(row0, STRIPE)])
  plsc.subcore_barrier()

  def step(k, carry):
    rb = rowbase0 + k * G
    pltpu.async_copy(dst2_hbm.at[pl.ds(rb, G)], didx, isem).wait()
    sd = [pltpu.async_copy(ones_v, acc.at[didx.at[g]], ssem, add=True)
          for g in range(G)]
    for d in sd:
      d.wait()
    return carry

  lax.fori_loop(0, NCHUNK // G, step, 0)
  plsc.subcore_barrier()
  pltpu.sync_copy(acc.at[pl.ds(row0, STRIPE)],
                  out_hbm.at[c, pl.ds(row0, STRIPE)])


_deg_call = pl.kernel(
    _deg_body,
    out_type=jax.ShapeDtypeStruct((NC, NP, FD), jnp.float32),
    mesh=_MESH,
    scratch_types=[
        pltpu.VMEM((G, CHUNK), jnp.int32),
        pltpu.VMEM((CHUNK, FD), jnp.float32),
        pltpu.VMEM_SHARED((NP, FD), jnp.float32),
        pltpu.SemaphoreType.DMA,
        pltpu.SemaphoreType.DMA,
    ],
    compiler_params=pltpu.CompilerParams(use_tc_tiling_on_sc=False),
    name="sc_degree",
)


def _blk_type(i):
  i = jnp.asarray(i)
  return ((i >= N_EV // BN).astype(jnp.int32)
          + (i >= (N_EV + N_CS) // BN).astype(jnp.int32)
          + (i >= (N_EV + N_CS + N_TR) // BN).astype(jnp.int32))


def _tc_embed_body(feat_ref, w_ref, b_ref, d_ref, dinv_ref, u1_ref):
  x = feat_ref[...]
  w = w_ref[0]
  b = b_ref[0]
  emb = jnp.dot(x, w, preferred_element_type=jnp.float32) + b
  x0 = jnp.maximum(emb, 0.0)
  d = d_ref[...]
  deg = d[0, :, 0:1] + d[1, :, 0:1] + 1.0
  dinv = lax.rsqrt(jnp.maximum(deg, 1.0))
  dinv_ref[...] = dinv
  u1_ref[...] = x0 * dinv


def _tc_dense1_body(p_ref, u1_ref, dinv_ref, w1_ref, b1_ref, w2_ref, u2_ref):
  p = p_ref[...]
  dinv = dinv_ref[...]
  agg = (p[0] + p[1] + u1_ref[...]) * dinv
  x1 = jnp.maximum(
      jnp.dot(agg, w1_ref[...], preferred_element_type=jnp.float32)
      + b1_ref[...], 0.0)
  u2_ref[...] = jnp.dot(
      x1, w2_ref[...], preferred_element_type=jnp.float32) * dinv


def _tc_dense2_body(q_ref, u2_ref, dinv_ref, b2_ref, wl_ref, u3_ref):
  q = q_ref[...]
  dinv = dinv_ref[...]
  x2 = jnp.maximum((q[0] + q[1] + u2_ref[...]) * dinv + b2_ref[...], 0.0)
  h3 = jnp.sum(x2 * wl_ref[...], axis=1, keepdims=True)
  col0 = (lax.broadcasted_iota(jnp.int32, (1, FD), 1) == 0)
  u3_ref[...] = (h3 * dinv) * col0.astype(jnp.float32)


def _tc_final_body(r_ref, u3_ref, dinv_ref, bl_ref, out_ref):
  r = r_ref[...]
  v = (r[0, :, 0:1] + r[1, :, 0:1] + u3_ref[:, 0:1]) * dinv_ref[...]
  out_ref[...] = MAX_ACTION * jnp.tanh(v + bl_ref[...])


def _full(shape):
  return pl.BlockSpec(shape, lambda i: (0,) * len(shape))


_row8 = pl.BlockSpec((BN, FD), lambda i: (i, 0))
_row1 = pl.BlockSpec((BN, 1), lambda i: (i, 0))
_par8 = pl.BlockSpec((NC, BN, FD), lambda i: (0, i, 0))

_tc_embed = pl.pallas_call(
    _tc_embed_body,
    grid=(GRID,),
    in_specs=[
        pl.BlockSpec((BN, 16), lambda i: (i, 0)),
        pl.BlockSpec((1, 16, FD), lambda i: (_blk_type(i), 0, 0)),
        pl.BlockSpec((1, 1, FD), lambda i: (_blk_type(i), 0, 0)),
        _par8,
    ],
    out_specs=[_row1, _row8],
    out_shape=[
        jax.ShapeDtypeStruct((NN, 1), jnp.float32),
        jax.ShapeDtypeStruct((NN, FD), jnp.float32),
    ],
)

_tc_dense1 = pl.pallas_call(
    _tc_dense1_body,
    grid=(GRID,),
    in_specs=[_par8, _row8, _row1, _full((FD, HD)), _full((1, HD)),
              _full((HD, FD))],
    out_specs=_row8,
    out_shape=jax.ShapeDtypeStruct((NN, FD), jnp.float32),
)

_tc_dense2 = pl.pallas_call(
    _tc_dense2_body,
    grid=(GRID,),
    in_specs=[_par8, _row8, _row1, _full((1, FD)), _full((1, FD))],
    out_specs=_row8,
    out_shape=jax.ShapeDtypeStruct((NN, FD), jnp.float32),
)

_tc_final = pl.pallas_call(
    _tc_final_body,
    grid=(GRID,),
    in_specs=[_par8, _row8, _row1, _full((1, 1))],
    out_specs=_row1,
    out_shape=jax.ShapeDtypeStruct((NN, 1), jnp.float32),
)


def kernel(ev_features, cs_features, tr_features, env_features, edge_index,
           ev_indexes, cs_indexes, tr_indexes, env_indexes,
           W_ev, b_ev, W_cs, b_cs, W_tr, b_tr, W_env, b_env,
           W_g1, b_g1, W_g2, b_g2, W_gl, b_gl):
  src = edge_index[0].reshape(NE // CHUNK, CHUNK)
  dst = edge_index[1].reshape(NE // CHUNK, CHUNK)

  feat = jnp.concatenate([
      ev_features,
      jnp.pad(cs_features, ((0, 0), (0, 4))),
      jnp.pad(tr_features, ((0, 0), (0, 6))),
      jnp.pad(env_features, ((0, 0), (0, 8))),
  ], axis=0)
  wstack = jnp.stack([
      W_ev,
      jnp.pad(W_cs, ((0, 4), (0, 0))),
      jnp.pad(W_tr, ((0, 6), (0, 0))),
      jnp.pad(W_env, ((0, 8), (0, 0))),
  ])
  bstack = jnp.stack([b_ev, b_cs, b_tr, b_env])[:, None, :]

  zeros = jnp.zeros((NP, FD), jnp.float32)
  ones_rows = jnp.zeros((CHUNK, FD), jnp.float32).at[:, 0].set(1.0)

  degp = _deg_call(dst, ones_rows, zeros)
  dinv, u1 = _tc_embed(feat, wstack, bstack, degp)
  p = _agg_call(src, dst, u1, zeros)
  u2 = _tc_dense1(p, u1, dinv, W_g1, b_g1[None, :], W_g2)
  q = _agg_call(src, dst, u2, zeros)
  u3 = _tc_dense2(q, u2, dinv, b_g2[None, :], W_gl.reshape(1, FD))
  r = _agg_call(src, dst, u3, zeros)
  out = _tc_final(r, u3, dinv, b_gl.reshape(1, 1))
  return out.reshape(-1)
